# Initial kernel scaffold; baseline (speedup 1.0000x reference)
import jax
import jax.numpy as jnp
from jax.experimental import pallas as pl


def kernel(x):
    raise NotImplementedError("write your pallas kernel here")


# speedup vs baseline: 6.1194x; 6.1194x over previous
import functools

import jax
import jax.numpy as jnp
from jax import lax
from jax.experimental import pallas as pl
from jax.experimental.pallas import tpu as pltpu
from jax.experimental.pallas import tpu_sc as plsc

L = 16
ROW = 1024
NB = ROW // L
NC, NS = 2, 16
NW = NC * NS
ROWS_TOTAL = 4 * 8192
ROWS_PER_W = ROWS_TOTAL // NW
RBLK = 16
GRP = 16


def _vsort(v):
    return lax.sort(v, dimension=0)


def _merge_tail(vals, dv0):
    n2 = len(vals)
    dv = dv0
    while dv >= 1:
        out = list(vals)
        for j in range(n2):
            if (j // dv) % 2 == 0:
                a, b = vals[j], vals[j + dv]
                out[j] = jnp.minimum(a, b)
                out[j + dv] = jnp.maximum(a, b)
        vals = out
        dv //= 2
    return [_vsort(v) for v in vals]


def _merge(ld, st, mv):
    n2 = 2 * mv
    if n2 <= GRP:
        vals = [ld(j) for j in range(n2)]
        out = [None] * n2
        for j in range(mv):
            a = vals[j]
            b = jnp.flip(vals[n2 - 1 - j], axis=0)
            out[j] = jnp.minimum(a, b)
            out[mv + j] = jnp.maximum(a, b)
        vals = _merge_tail(out, mv // 2)
        for j in range(n2):
            st(j, vals[j])
        return

    for j in range(mv // 2):
        jj = mv - 1 - j
        a0 = ld(j)
        b0 = jnp.flip(ld(n2 - 1 - j), axis=0)
        a1 = ld(jj)
        b1 = jnp.flip(ld(mv + j), axis=0)
        st(j, jnp.minimum(a0, b0))
        st(mv + j, jnp.maximum(a0, b0))
        st(jj, jnp.minimum(a1, b1))
        st(mv + jj, jnp.maximum(a1, b1))

    dv = mv // 2
    while dv >= GRP:
        for j in range(n2):
            if (j // dv) % 2 == 0:
                a = ld(j)
                b = ld(j + dv)
                st(j, jnp.minimum(a, b))
                st(j + dv, jnp.maximum(a, b))
        dv //= 2

    for g in range(n2 // GRP):
        vals = [ld(g * GRP + j) for j in range(GRP)]
        vals = _merge_tail(vals, GRP // 2)
        for j in range(GRP):
            st(g * GRP + j, vals[j])


def _make_kernel():
    mesh = plsc.VectorSubcoreMesh(core_axis_name="c", subcore_axis_name="s")

    @functools.partial(
        pl.kernel,
        out_type=jax.ShapeDtypeStruct((ROWS_TOTAL, ROW), jnp.float32),
        mesh=mesh,
        scratch_types=[pltpu.VMEM((RBLK, ROW), jnp.float32)],
        compiler_params=pltpu.CompilerParams(needs_layout_passes=False),
    )
    def sort_kernel(x_hbm, out_hbm, buf):
        wid = lax.axis_index("s") * NC + lax.axis_index("c")
        row0 = wid * ROWS_PER_W

        def chunk_body(ci, carry):
            r0 = row0 + ci * RBLK
            pltpu.sync_copy(x_hbm.at[pl.ds(r0, RBLK)], buf)

            def row_body(r, carry2):
                def ld_at(base):
                    def ld(j):
                        return buf[r, pl.ds(base + j * L, L)]
                    return ld

                def st_at(base):
                    def st(j, v):
                        buf[r, pl.ds(base + j * L, L)] = v
                    return st

                def init_body(j, c3):
                    v = buf[r, pl.ds(j * L, L)]
                    buf[r, pl.ds(j * L, L)] = _vsort(v)
                    return c3

                lax.fori_loop(0, NB, init_body, 0, unroll=16)

                unrolls = {16: 8, 32: 4, 64: 2, 128: 2, 256: 2, 512: 1}
                m = L
                while m < ROW:
                    mv = m // L
                    nmerge = ROW // (2 * m)

                    def merge_body(t, c3, mv=mv, m=m):
                        base = t * 2 * m
                        _merge(ld_at(base), st_at(base), mv)
                        return c3

                    lax.fori_loop(0, nmerge, merge_body, 0,
                                  unroll=unrolls[m])
                    m *= 2
                return carry2

            lax.fori_loop(0, RBLK, row_body, 0)
            pltpu.sync_copy(buf, out_hbm.at[pl.ds(r0, RBLK)])
            return carry

        lax.fori_loop(0, ROWS_PER_W // RBLK, chunk_body, 0)

    return sort_kernel


_sort_rows = _make_kernel()


def kernel(x):
    b, s, n = x.shape
    out = _sort_rows(x.reshape(b * s, n))
    return out.reshape(b, s, n)


# speedup vs baseline: 6.5364x; 1.0682x over previous
import functools

import jax
import jax.numpy as jnp
from jax import lax
from jax.experimental import pallas as pl
from jax.experimental.pallas import tpu as pltpu
from jax.experimental.pallas import tpu_sc as plsc

L = 16
ROW = 1024
NB = ROW // L
NC, NS = 2, 16
NW = NC * NS
ROWS_TOTAL = 4 * 8192
ROWS_PER_W = ROWS_TOTAL // NW
RBLK = 16
GRP = 16


def _vsort(v):
    return lax.sort(v, dimension=0)


def _merge_tail(vals, dv0):
    n2 = len(vals)
    dv = dv0
    while dv >= 1:
        out = list(vals)
        for j in range(n2):
            if (j // dv) % 2 == 0:
                a, b = vals[j], vals[j + dv]
                out[j] = jnp.minimum(a, b)
                out[j + dv] = jnp.maximum(a, b)
        vals = out
        dv //= 2
    return [_vsort(v) for v in vals]


def _merge_vals(vals):
    n2 = len(vals)
    mv = n2 // 2
    out = [None] * n2
    for j in range(mv):
        a = vals[j]
        b = jnp.flip(vals[n2 - 1 - j], axis=0)
        out[j] = jnp.minimum(a, b)
        out[mv + j] = jnp.maximum(a, b)
    return _merge_tail(out, mv // 2)


def _sort_group(vals):
    vals = [_vsort(v) for v in vals]
    mb = 1
    while mb < len(vals):
        out = []
        for t in range(len(vals) // (2 * mb)):
            out.extend(_merge_vals(vals[t * 2 * mb:(t + 1) * 2 * mb]))
        vals = out
        mb *= 2
    return vals


def _merge(ld, st, mv):
    n2 = 2 * mv
    if n2 <= GRP:
        vals = _merge_vals([ld(j) for j in range(n2)])
        for j in range(n2):
            st(j, vals[j])
        return

    for j in range(mv // 2):
        jj = mv - 1 - j
        a0 = ld(j)
        b0 = jnp.flip(ld(n2 - 1 - j), axis=0)
        a1 = ld(jj)
        b1 = jnp.flip(ld(mv + j), axis=0)
        st(j, jnp.minimum(a0, b0))
        st(mv + j, jnp.maximum(a0, b0))
        st(jj, jnp.minimum(a1, b1))
        st(mv + jj, jnp.maximum(a1, b1))

    dv = mv // 2
    while dv >= GRP:
        for j in range(n2):
            if (j // dv) % 2 == 0:
                a = ld(j)
                b = ld(j + dv)
                st(j, jnp.minimum(a, b))
                st(j + dv, jnp.maximum(a, b))
        dv //= 2

    for g in range(n2 // GRP):
        vals = [ld(g * GRP + j) for j in range(GRP)]
        vals = _merge_tail(vals, GRP // 2)
        for j in range(GRP):
            st(g * GRP + j, vals[j])


def _make_kernel():
    mesh = plsc.VectorSubcoreMesh(core_axis_name="c", subcore_axis_name="s")

    @functools.partial(
        pl.kernel,
        out_type=jax.ShapeDtypeStruct((ROWS_TOTAL, ROW), jnp.float32),
        mesh=mesh,
        scratch_types=[pltpu.VMEM((RBLK, ROW), jnp.float32)],
        compiler_params=pltpu.CompilerParams(needs_layout_passes=False),
    )
    def sort_kernel(x_hbm, out_hbm, buf):
        wid = lax.axis_index("s") * NC + lax.axis_index("c")
        row0 = wid * ROWS_PER_W

        def chunk_body(ci, carry):
            r0 = row0 + ci * RBLK
            pltpu.sync_copy(x_hbm.at[pl.ds(r0, RBLK)], buf)

            def row_body(r, carry2):
                def ld_at(base):
                    def ld(j):
                        return buf[r, pl.ds(base + j * L, L)]
                    return ld

                def st_at(base):
                    def st(j, v):
                        buf[r, pl.ds(base + j * L, L)] = v
                    return st

                def group_body(g, c3):
                    base = g * GRP * L
                    ld = ld_at(base)
                    st = st_at(base)
                    vals = _sort_group([ld(j) for j in range(GRP)])
                    for j in range(GRP):
                        st(j, vals[j])
                    return c3

                lax.fori_loop(0, NB // GRP, group_body, 0)

                m = GRP * L
                while m < ROW:
                    mv = m // L
                    nmerge = ROW // (2 * m)

                    def merge_body(t, c3, mv=mv, m=m):
                        base = t * 2 * m
                        _merge(ld_at(base), st_at(base), mv)
                        return c3

                    lax.fori_loop(0, nmerge, merge_body, 0,
                                  unroll=max(1, nmerge))
                    m *= 2
                return carry2

            lax.fori_loop(0, RBLK, row_body, 0)
            pltpu.sync_copy(buf, out_hbm.at[pl.ds(r0, RBLK)])
            return carry

        lax.fori_loop(0, ROWS_PER_W // RBLK, chunk_body, 0)

    return sort_kernel


_sort_rows = _make_kernel()


def kernel(x):
    b, s, n = x.shape
    out = _sort_rows(x.reshape(b * s, n))
    return out.reshape(b, s, n)


# speedup vs baseline: 6.8120x; 1.0422x over previous
import functools

import jax
import jax.numpy as jnp
from jax import lax
from jax.experimental import pallas as pl
from jax.experimental.pallas import tpu as pltpu
from jax.experimental.pallas import tpu_sc as plsc

L = 16
ROW = 1024
NB = ROW // L
NC, NS = 2, 16
NW = NC * NS
ROWS_TOTAL = 4 * 8192
ROWS_PER_W = ROWS_TOTAL // NW
RBLK = 16
GRP = 16


def _vsort(v, desc=False):
    if desc:
        return plsc.sort_key_val(v, v, descending=True)[0]
    return lax.sort(v, dimension=0)


def _minmax(a, b, desc):
    if desc:
        return jnp.maximum(a, b), jnp.minimum(a, b)
    return jnp.minimum(a, b), jnp.maximum(a, b)


def _merge_tail(vals, dv0, desc):
    n2 = len(vals)
    dv = dv0
    while dv >= 1:
        out = list(vals)
        for j in range(n2):
            if (j // dv) % 2 == 0:
                out[j], out[j + dv] = _minmax(vals[j], vals[j + dv], desc)
        vals = out
        dv //= 2
    return [_vsort(v, desc) for v in vals]


def _merge_vals(vals, desc):
    n2 = len(vals)
    mv = n2 // 2
    out = [None] * n2
    for j in range(mv):
        out[j], out[mv + j] = _minmax(vals[j], vals[mv + j], desc)
    return _merge_tail(out, mv // 2, desc)


def _sort_group(vals, final_desc):
    n = len(vals)
    vals = [_vsort(v, desc=(j % 2 == 1)) for j, v in enumerate(vals)]
    mb = 1
    while mb < n:
        out = []
        for t in range(n // (2 * mb)):
            d = final_desc if 2 * mb == n else (t % 2 == 1)
            out.extend(_merge_vals(vals[t * 2 * mb:(t + 1) * 2 * mb], d))
        vals = out
        mb *= 2
    return vals


def _merge_mem(ld, st, mv, desc):
    n2 = 2 * mv
    if n2 <= GRP:
        vals = _merge_vals([ld(j) for j in range(n2)], desc)
        for j in range(n2):
            st(j, vals[j])
        return

    for j in range(mv):
        lo, hi = _minmax(ld(j), ld(mv + j), desc)
        st(j, lo)
        st(mv + j, hi)

    dv = mv // 2
    while dv >= GRP:
        for j in range(n2):
            if (j // dv) % 2 == 0:
                lo, hi = _minmax(ld(j), ld(j + dv), desc)
                st(j, lo)
                st(j + dv, hi)
        dv //= 2

    for g in range(n2 // GRP):
        vals = [ld(g * GRP + j) for j in range(GRP)]
        vals = _merge_tail(vals, GRP // 2, desc)
        for j in range(GRP):
            st(g * GRP + j, vals[j])


def _make_kernel():
    mesh = plsc.VectorSubcoreMesh(core_axis_name="c", subcore_axis_name="s")

    @functools.partial(
        pl.kernel,
        out_type=jax.ShapeDtypeStruct((ROWS_TOTAL, ROW), jnp.float32),
        mesh=mesh,
        scratch_types=[pltpu.VMEM((RBLK, ROW), jnp.float32)],
        compiler_params=pltpu.CompilerParams(needs_layout_passes=False),
    )
    def sort_kernel(x_hbm, out_hbm, buf):
        wid = lax.axis_index("s") * NC + lax.axis_index("c")
        row0 = wid * ROWS_PER_W

        def chunk_body(ci, carry):
            r0 = row0 + ci * RBLK
            pltpu.sync_copy(x_hbm.at[pl.ds(r0, RBLK)], buf)

            def row_body(r, carry2):
                def ld_at(base):
                    def ld(j):
                        return buf[r, pl.ds(base + j * L, L)]
                    return ld

                def st_at(base):
                    def st(j, v):
                        buf[r, pl.ds(base + j * L, L)] = v
                    return st

                def gpair_body(p, c3):
                    for gb in range(2):
                        base = (2 * p + gb) * GRP * L
                        ld = ld_at(base)
                        st = st_at(base)
                        vals = _sort_group([ld(j) for j in range(GRP)],
                                           final_desc=(gb == 1))
                        for j in range(GRP):
                            st(j, vals[j])
                    return c3

                lax.fori_loop(0, NB // GRP // 2, gpair_body, 0)

                _merge_mem(ld_at(0), st_at(0), 16, False)
                _merge_mem(ld_at(512), st_at(512), 16, True)
                _merge_mem(ld_at(0), st_at(0), 32, False)
                return carry2

            lax.fori_loop(0, RBLK, row_body, 0)
            pltpu.sync_copy(buf, out_hbm.at[pl.ds(r0, RBLK)])
            return carry

        lax.fori_loop(0, ROWS_PER_W // RBLK, chunk_body, 0)

    return sort_kernel


# speedup vs baseline: 7.8130x; 1.1470x over previous
import functools

import jax
import jax.numpy as jnp
from jax import lax
from jax.experimental import pallas as pl
from jax.experimental.pallas import tpu as pltpu
from jax.experimental.pallas import tpu_sc as plsc

L = 16
ROW = 1024
NB = ROW // L
NC, NS = 2, 16
NW = NC * NS
ROWS_TOTAL = 4 * 8192
ROWS_PER_W = ROWS_TOTAL // NW
RBLK = 16
GRP = 16


def _vsort(v, desc=False):
    if desc:
        return plsc.sort_key_val(v, v, descending=True)[0]
    return lax.sort(v, dimension=0)


def _minmax(a, b, desc):
    if desc:
        return jnp.maximum(a, b), jnp.minimum(a, b)
    return jnp.minimum(a, b), jnp.maximum(a, b)


def _merge_tail(vals, dv0, desc):
    n2 = len(vals)
    dv = dv0
    while dv >= 1:
        out = list(vals)
        for j in range(n2):
            if (j // dv) % 2 == 0:
                out[j], out[j + dv] = _minmax(vals[j], vals[j + dv], desc)
        vals = out
        dv //= 2
    return [_vsort(v, desc) for v in vals]


def _merge_vals(vals, desc):
    n2 = len(vals)
    mv = n2 // 2
    out = [None] * n2
    for j in range(mv):
        out[j], out[mv + j] = _minmax(vals[j], vals[mv + j], desc)
    return _merge_tail(out, mv // 2, desc)


def _sort_group(vals, final_desc):
    n = len(vals)
    vals = [_vsort(v, desc=(j % 2 == 1)) for j, v in enumerate(vals)]
    mb = 1
    while mb < n:
        out = []
        for t in range(n // (2 * mb)):
            d = final_desc if 2 * mb == n else (t % 2 == 1)
            out.extend(_merge_vals(vals[t * 2 * mb:(t + 1) * 2 * mb], d))
        vals = out
        mb *= 2
    return vals


def _merge_mem(ld, st, mv, desc):
    n2 = 2 * mv
    if n2 <= GRP:
        vals = _merge_vals([ld(j) for j in range(n2)], desc)
        for j in range(n2):
            st(j, vals[j])
        return

    for j in range(mv):
        lo, hi = _minmax(ld(j), ld(mv + j), desc)
        st(j, lo)
        st(mv + j, hi)

    dv = mv // 2
    while dv >= GRP:
        for j in range(n2):
            if (j // dv) % 2 == 0:
                lo, hi = _minmax(ld(j), ld(j + dv), desc)
                st(j, lo)
                st(j + dv, hi)
        dv //= 2

    for g in range(n2 // GRP):
        vals = [ld(g * GRP + j) for j in range(GRP)]
        vals = _merge_tail(vals, GRP // 2, desc)
        for j in range(GRP):
            st(g * GRP + j, vals[j])


RING = 4
NCHUNK = ROWS_PER_W // RBLK


def _make_kernel():
    mesh = plsc.VectorSubcoreMesh(core_axis_name="c", subcore_axis_name="s")

    @functools.partial(
        pl.kernel,
        out_type=jax.ShapeDtypeStruct((ROWS_TOTAL, ROW), jnp.float32),
        mesh=mesh,
        scratch_types=(
            [pltpu.VMEM((RBLK, ROW), jnp.float32)] * RING
            + [pltpu.SemaphoreType.DMA] * (2 * RING)
        ),
        compiler_params=pltpu.CompilerParams(needs_layout_passes=False),
    )
    def sort_kernel(x_hbm, out_hbm, *refs):
        bufs = refs[:RING]
        isems = refs[RING:2 * RING]
        osems = refs[2 * RING:]
        wid = lax.axis_index("s") * NC + lax.axis_index("c")
        row0 = wid * ROWS_PER_W

        def in_copy(ci, b):
            pltpu.async_copy(
                x_hbm.at[pl.ds(row0 + ci * RBLK, RBLK)], bufs[b], isems[b])

        def wait_in(b):
            pltpu.make_async_copy(
                x_hbm.at[pl.ds(row0, RBLK)], bufs[b], isems[b]).wait()

        def out_copy(ci, b):
            pltpu.async_copy(
                bufs[b], out_hbm.at[pl.ds(row0 + ci * RBLK, RBLK)], osems[b])

        def wait_out(b):
            pltpu.make_async_copy(
                bufs[b], out_hbm.at[pl.ds(row0, RBLK)], osems[b]).wait()

        def sort_chunk(buf):
            def row_body(r, carry2):
                def ld_at(base):
                    def ld(j):
                        return buf[r, pl.ds(base + j * L, L)]
                    return ld

                def st_at(base):
                    def st(j, v):
                        buf[r, pl.ds(base + j * L, L)] = v
                    return st

                def gpair_body(p, c3):
                    for gb in range(2):
                        base = (2 * p + gb) * GRP * L
                        ld = ld_at(base)
                        st = st_at(base)
                        vals = _sort_group([ld(j) for j in range(GRP)],
                                           final_desc=(gb == 1))
                        for j in range(GRP):
                            st(j, vals[j])
                    return c3

                lax.fori_loop(0, NB // GRP // 2, gpair_body, 0)

                _merge_mem(ld_at(0), st_at(0), 16, False)
                _merge_mem(ld_at(512), st_at(512), 16, True)
                _merge_mem(ld_at(0), st_at(0), 32, False)
                return carry2

            lax.fori_loop(0, RBLK, row_body, 0)

        in_copy(0, 0)
        in_copy(1, 1)
        out_copy(2, 2)
        out_copy(3, 3)

        def ring_body(p, carry):
            for b in range(RING):
                ci = p * RING + b
                wait_in(b)
                nb = (b + 2) % RING
                wait_out(nb)
                in_copy(jnp.minimum(ci + 2, NCHUNK - 1), nb)
                sort_chunk(bufs[b])
                out_copy(ci, b)
            return carry

        lax.fori_loop(0, NCHUNK // RING, ring_body, 0)

        wait_in(0)
        wait_in(1)
        wait_out(2)
        wait_out(3)

    return sort_kernel


_sort_rows = _make_kernel()


def kernel(x):
    b, s, n = x.shape
    out = _sort_rows(x.reshape(b * s, n))
    return out.reshape(b, s, n)
